# Initial kernel scaffold; baseline (speedup 1.0000x reference)
import jax
import jax.numpy as jnp
from jax.experimental import pallas as pl


def kernel(x, edge_index, init_weight, root_weight, bias):
    raise NotImplementedError("write your pallas kernel here")


# speedup vs baseline: 10.1929x; 10.1929x over previous
import dataclasses
import functools

import jax
import jax.numpy as jnp
from jax import lax
from jax.experimental import pallas as pl
from jax.experimental.pallas import tpu as pltpu
from jax.experimental.pallas import tpu_sc as plsc

N = 10000
E = 320000
F = 128

NC = 2
NS = 16
NW = NC * NS

EPT = E // NW
NPAD = 10240
SLICE = NPAD // NS

_vector_mesh = plsc.VectorSubcoreMesh(
    core_axis_name="c", subcore_axis_name="s")

_sc_params = pltpu.CompilerParams()
if "needs_layout_passes" in pltpu.CompilerParams.__dataclass_fields__:
    _sc_params = dataclasses.replace(_sc_params, needs_layout_passes=False)


@functools.partial(
    pl.kernel,
    out_type=jax.ShapeDtypeStruct((NC, NPAD), jnp.float32),
    mesh=_vector_mesh,
    compiler_params=_sc_params,
    scratch_types=[
        pltpu.VMEM((NPAD,), jnp.float32),
        pltpu.VMEM((EPT,), jnp.int32),
        pltpu.VMEM((SLICE,), jnp.float32),
        pltpu.VMEM((SLICE,), jnp.float32),
        pltpu.VMEM_SHARED((NS, NPAD), jnp.float32),
    ],
)
def _sc_degree(col_hbm, out_hbm, hist_v, cidx_v, acc_v, tmp_v, stage_sp):
    cid = lax.axis_index("c")
    sid = lax.axis_index("s")
    wid = cid * NS + sid

    @pl.loop(0, NPAD, step=16)
    def _zero(i):
        hist_v[pl.ds(i, 16)] = jnp.zeros((16,), jnp.float32)

    pltpu.sync_copy(col_hbm.at[pl.ds(wid * EPT, EPT)], cidx_v)
    ones = jnp.ones((16,), jnp.float32)

    @pl.loop(0, EPT, step=16)
    def _hist(i):
        idx = cidx_v[pl.ds(i, 16)]
        plsc.addupdate_scatter(hist_v, [idx], ones)

    pltpu.sync_copy(hist_v, stage_sp.at[sid])
    plsc.subcore_barrier()

    base = sid * SLICE
    pltpu.sync_copy(stage_sp.at[0, pl.ds(base, SLICE)], acc_v)

    @pl.loop(1, NS)
    def _merge(k):
        pltpu.sync_copy(stage_sp.at[k, pl.ds(base, SLICE)], tmp_v)

        @pl.loop(0, SLICE, step=16)
        def _add(j):
            acc_v[pl.ds(j, 16)] = acc_v[pl.ds(j, 16)] + tmp_v[pl.ds(j, 16)]

    pltpu.sync_copy(acc_v, out_hbm.at[cid, pl.ds(base, SLICE)])


ECH = 80
EPC = E // NS
NIT = EPC // ECH
NHALF = 5120
NP2 = 2 * NHALF
TRASH = NHALF
RPT = NHALF // NS


@functools.partial(
    pl.kernel,
    out_type=jax.ShapeDtypeStruct((NP2, F), jnp.float32),
    mesh=_vector_mesh,
    compiler_params=_sc_params,
    scratch_types=[
        pltpu.VMEM((ECH,), jnp.int32),
        pltpu.VMEM((ECH,), jnp.int32),
        pltpu.VMEM((ECH, F), jnp.float32),
        pltpu.VMEM((RPT, F), jnp.float32),
        pltpu.VMEM_SHARED((NHALF + 8, F), jnp.float32),
    ],
)
def _sc_propagate(h2_hbm, row_hbm, col_hbm, out_hbm,
                  ridx_v, cidx_v, rows_v, buf_v, acc_sp):
    cid = lax.axis_index("c")
    sid = lax.axis_index("s")
    ebase = sid * EPC
    rbase = sid * RPT
    lo = cid * NHALF

    @pl.loop(0, RPT)
    def _zero(r):
        @pl.loop(0, F, step=16)
        def _z16(q):
            buf_v[r, pl.ds(q, 16)] = jnp.zeros((16,), jnp.float32)

    pltpu.sync_copy(buf_v, acc_sp.at[pl.ds(rbase, RPT)])

    @pl.when(sid == 0)
    def _zero_trash():
        pltpu.sync_copy(buf_v.at[pl.ds(0, 8)], acc_sp.at[pl.ds(NHALF, 8)])

    plsc.subcore_barrier()

    @pl.loop(0, NIT)
    def _edges(j):
        e0 = ebase + j * ECH
        pltpu.sync_copy(row_hbm.at[pl.ds(e0, ECH)], ridx_v)
        pltpu.sync_copy(col_hbm.at[pl.ds(e0, ECH)], cidx_v)
        pltpu.sync_copy(h2_hbm.at[ridx_v], rows_v)
        for q in range(ECH // 16):
            c16 = cidx_v[pl.ds(q * 16, 16)]
            loc = c16 - lo
            ok = (loc >= 0) & (loc < NHALF)
            cidx_v[pl.ds(q * 16, 16)] = jnp.where(ok, loc, TRASH)
        pltpu.sync_copy(rows_v, acc_sp.at[cidx_v], add=True)

    plsc.subcore_barrier()
    pltpu.sync_copy(acc_sp.at[pl.ds(rbase, RPT)], buf_v)
    pltpu.sync_copy(buf_v, out_hbm.at[pl.ds(lo + rbase, RPT)])


_RB = 1000
_NB = N // _RB


def _tc_mm_body(x_ref, dg_ref, wi_ref, wr_ref, b_ref, h2_ref, root_ref):
    d = dg_ref[0] + dg_ref[1]
    dis = jnp.where(d > 0, lax.rsqrt(d), 0.0)
    xb = x_ref[...]
    h2_ref[...] = jnp.dot(xb, wi_ref[...],
                          preferred_element_type=jnp.float32) * dis
    root_ref[...] = jnp.dot(xb, wr_ref[...],
                            preferred_element_type=jnp.float32) + b_ref[...]


def _tc_matmuls(x, degp, wi, wr, b2):
    return pl.pallas_call(
        _tc_mm_body,
        grid=(_NB,),
        in_specs=[
            pl.BlockSpec((_RB, F), lambda i: (i, 0)),
            pl.BlockSpec((NC, _RB, 1), lambda i: (0, i, 0)),
            pl.BlockSpec((F, F), lambda i: (0, 0)),
            pl.BlockSpec((F, F), lambda i: (0, 0)),
            pl.BlockSpec((1, F), lambda i: (0, 0)),
        ],
        out_specs=[
            pl.BlockSpec((_RB, F), lambda i: (i, 0)),
            pl.BlockSpec((_RB, F), lambda i: (i, 0)),
        ],
        out_shape=[
            jax.ShapeDtypeStruct((N, F), jnp.float32),
            jax.ShapeDtypeStruct((N, F), jnp.float32),
        ],
    )(x, degp, wi, wr, b2)


def _tc_fin_body(p_ref, dg_ref, root_ref, o_ref):
    d = dg_ref[0] + dg_ref[1]
    dis = jnp.where(d > 0, lax.rsqrt(d), 0.0)
    s = p_ref[...] * dis + root_ref[...]
    o_ref[...] = jnp.maximum(s, 0.0)


def _tc_finish(parts, degp, root):
    return pl.pallas_call(
        _tc_fin_body,
        grid=(_NB,),
        in_specs=[
            pl.BlockSpec((_RB, F), lambda i: (i, 0)),
            pl.BlockSpec((NC, _RB, 1), lambda i: (0, i, 0)),
            pl.BlockSpec((_RB, F), lambda i: (i, 0)),
        ],
        out_specs=pl.BlockSpec((_RB, F), lambda i: (i, 0)),
        out_shape=jax.ShapeDtypeStruct((N, F), jnp.float32),
    )(parts, degp, root)


def kernel(x, edge_index, init_weight, root_weight, bias):
    row = edge_index[0]
    col = edge_index[1]
    deg2 = _sc_degree(col)
    degp = deg2[:, :N, None]
    b2 = bias[0, 0]
    h2, root = _tc_matmuls(x, degp, init_weight[0], root_weight[0, 0], b2)
    parts = _sc_propagate(h2, row, col)
    return _tc_finish(parts, degp, root)


# speedup vs baseline: 11.8604x; 1.1636x over previous
import dataclasses
import functools

import jax
import jax.numpy as jnp
from jax import lax
from jax.experimental import pallas as pl
from jax.experimental.pallas import tpu as pltpu
from jax.experimental.pallas import tpu_sc as plsc

N = 10000
E = 320000
F = 128

NC = 2
NS = 16
NW = NC * NS

EPT = E // NW
NPAD = 10240
SLICE = NPAD // NS

_vector_mesh = plsc.VectorSubcoreMesh(
    core_axis_name="c", subcore_axis_name="s")

_sc_params = pltpu.CompilerParams()
if "needs_layout_passes" in pltpu.CompilerParams.__dataclass_fields__:
    _sc_params = dataclasses.replace(_sc_params, needs_layout_passes=False)


@functools.partial(
    pl.kernel,
    out_type=[
        jax.ShapeDtypeStruct((NC, NPAD), jnp.float32),
        jax.ShapeDtypeStruct((NC, NS, NPAD), jnp.float32),
    ],
    mesh=_vector_mesh,
    compiler_params=_sc_params,
    scratch_types=[
        pltpu.VMEM((NPAD,), jnp.float32),
        pltpu.VMEM((EPT,), jnp.int32),
        pltpu.VMEM((SLICE,), jnp.float32),
        pltpu.VMEM((NS, SLICE), jnp.float32),
    ],
)
def _sc_degree(col_hbm, out_hbm, stage_hbm, hist_v, cidx_v, acc_v, tmp_v):
    cid = lax.axis_index("c")
    sid = lax.axis_index("s")
    wid = cid * NS + sid

    @pl.loop(0, NPAD, step=16)
    def _zero(i):
        hist_v[pl.ds(i, 16)] = jnp.zeros((16,), jnp.float32)

    pltpu.sync_copy(col_hbm.at[pl.ds(wid * EPT, EPT)], cidx_v)
    ones = jnp.ones((16,), jnp.float32)

    @pl.loop(0, EPT, step=16)
    def _hist(i):
        idx = cidx_v[pl.ds(i, 16)]
        plsc.addupdate_scatter(hist_v, [idx], ones)

    pltpu.sync_copy(hist_v, stage_hbm.at[cid, sid])
    plsc.subcore_barrier()

    base = sid * SLICE
    pltpu.sync_copy(stage_hbm.at[cid, :, pl.ds(base, SLICE)], tmp_v)

    @pl.loop(0, SLICE, step=16)
    def _add(j):
        s = tmp_v[0, pl.ds(j, 16)]
        for k in range(1, NS):
            s = s + tmp_v[k, pl.ds(j, 16)]
        acc_v[pl.ds(j, 16)] = s

    pltpu.sync_copy(acc_v, out_hbm.at[cid, pl.ds(base, SLICE)])


ECH = 128
EPC = E // NS
SCH = 4000
NSC = EPC // SCH
KMAX = SCH + 256
NHALF = 5120
NP2 = 2 * NHALF
TRASH = NHALF
RPT = NHALF // NS
DB = 160


@functools.partial(
    pl.kernel,
    out_type=jax.ShapeDtypeStruct((NP2, F), jnp.float32),
    mesh=_vector_mesh,
    compiler_params=_sc_params,
    scratch_types=[
        pltpu.VMEM((KMAX,), jnp.int32),
        pltpu.VMEM((KMAX,), jnp.int32),
        pltpu.VMEM((SCH,), jnp.int32),
        pltpu.VMEM((SCH,), jnp.int32),
        pltpu.VMEM((2, ECH, F), jnp.float32),
        pltpu.VMEM((ECH,), jnp.int32),
        pltpu.VMEM((DB, F), jnp.float32),
        pltpu.SemaphoreType.DMA,
        pltpu.SemaphoreType.DMA,
        pltpu.VMEM_SHARED((NHALF + 8, F), jnp.float32),
    ],
)
def _sc_propagate(h2_hbm, row_hbm, col_hbm, out_hbm,
                  rkeep, ckeep, escan_r, escan_c, rows_v, sidx_v, dump_v,
                  sg0, sg1, acc_sp):
    cid = lax.axis_index("c")
    sid = lax.axis_index("s")
    ebase = sid * EPC
    rbase = sid * RPT
    lo = cid * NHALF

    @pl.loop(0, DB)
    def _zr(r):
        @pl.loop(0, F, step=16)
        def _zq(q):
            dump_v[r, pl.ds(q, 16)] = jnp.zeros((16,), jnp.float32)

    pltpu.sync_copy(dump_v, acc_sp.at[pl.ds(rbase, DB)])
    pltpu.sync_copy(dump_v, acc_sp.at[pl.ds(rbase + DB, DB)])

    @pl.when(sid == 0)
    def _zero_trash():
        pltpu.sync_copy(dump_v.at[pl.ds(0, 8)], acc_sp.at[pl.ds(NHALF, 8)])

    plsc.subcore_barrier()

    for p in range(NSC):
        e0 = ebase + p * SCH
        pltpu.sync_copy(row_hbm.at[pl.ds(e0, SCH)], escan_r)
        pltpu.sync_copy(col_hbm.at[pl.ds(e0, SCH)], escan_c)

        def _scan(i, off):
            r16 = escan_r[pl.ds(i, 16)]
            c16 = escan_c[pl.ds(i, 16)]
            loc = c16 - lo
            ok = (loc >= 0) & (loc < NHALF)
            cs = plsc.cumsum(jnp.where(ok, 1, 0).astype(jnp.int32))
            pos = off + cs - 1
            plsc.store_scatter(rkeep, [pos], r16, mask=ok)
            plsc.store_scatter(ckeep, [pos], loc, mask=ok)
            return off + jnp.max(cs)

        kc = pl.loop(0, SCH, step=16, init_carry=jnp.int32(0))(_scan)

        @pl.loop(0, 256, step=16)
        def _pad(t):
            idx16 = kc + t + lax.iota(jnp.int32, 16)
            plsc.store_scatter(rkeep, [idx16], jnp.zeros((16,), jnp.int32))
            plsc.store_scatter(ckeep, [idx16],
                               jnp.full((16,), TRASH, jnp.int32))

        nch = (kc + (ECH - 1)) // ECH

        @pl.when(nch > 0)
        def _prologue():
            pltpu.async_copy(h2_hbm.at[rkeep.at[pl.ds(0, ECH)]],
                             rows_v.at[0], sg0)

        def _item(item, b, sg, sg_o):
            for q in range(ECH // 16):
                sidx_v[pl.ds(q * 16, 16)] = ckeep[pl.ds(item * ECH + q * 16,
                                                        16)]
            pltpu.make_async_copy(
                h2_hbm.at[rkeep.at[pl.ds(item * ECH, ECH)]],
                rows_v.at[b], sg).wait()

            @pl.when(item + 1 < nch)
            def _next():
                pltpu.async_copy(
                    h2_hbm.at[rkeep.at[pl.ds((item + 1) * ECH, ECH)]],
                    rows_v.at[1 - b], sg_o)

            pltpu.sync_copy(rows_v.at[b], acc_sp.at[sidx_v], add=True)

        @pl.loop(0, nch, step=2)
        def _phase_b(j):
            _item(j, 0, sg0, sg1)

            @pl.when(j + 1 < nch)
            def _odd():
                _item(j + 1, 1, sg1, sg0)

    plsc.subcore_barrier()
    for t in range(2):
        pltpu.sync_copy(acc_sp.at[pl.ds(rbase + t * DB, DB)], dump_v)
        pltpu.sync_copy(dump_v, out_hbm.at[pl.ds(lo + rbase + t * DB, DB)])


_RB = 1000
_NB = N // _RB


def _tc_mm_body(x_ref, dg_ref, wi_ref, wr_ref, b_ref, h2_ref, root_ref):
    d = dg_ref[0] + dg_ref[1]
    dis = jnp.where(d > 0, lax.rsqrt(d), 0.0)
    xb = x_ref[...]
    h2_ref[...] = jnp.dot(xb, wi_ref[...],
                          preferred_element_type=jnp.float32) * dis
    root_ref[...] = jnp.dot(xb, wr_ref[...],
                            preferred_element_type=jnp.float32) + b_ref[...]


def _tc_matmuls(x, degp, wi, wr, b2):
    return pl.pallas_call(
        _tc_mm_body,
        grid=(_NB,),
        in_specs=[
            pl.BlockSpec((_RB, F), lambda i: (i, 0)),
            pl.BlockSpec((NC, _RB, 1), lambda i: (0, i, 0)),
            pl.BlockSpec((F, F), lambda i: (0, 0)),
            pl.BlockSpec((F, F), lambda i: (0, 0)),
            pl.BlockSpec((1, F), lambda i: (0, 0)),
        ],
        out_specs=[
            pl.BlockSpec((_RB, F), lambda i: (i, 0)),
            pl.BlockSpec((_RB, F), lambda i: (i, 0)),
        ],
        out_shape=[
            jax.ShapeDtypeStruct((N, F), jnp.float32),
            jax.ShapeDtypeStruct((N, F), jnp.float32),
        ],
    )(x, degp, wi, wr, b2)


def _tc_fin_body(p_ref, dg_ref, root_ref, o_ref):
    d = dg_ref[0] + dg_ref[1]
    dis = jnp.where(d > 0, lax.rsqrt(d), 0.0)
    s = p_ref[...] * dis + root_ref[...]
    o_ref[...] = jnp.maximum(s, 0.0)


def _tc_finish(parts, degp, root):
    return pl.pallas_call(
        _tc_fin_body,
        grid=(_NB,),
        in_specs=[
            pl.BlockSpec((_RB, F), lambda i: (i, 0)),
            pl.BlockSpec((NC, _RB, 1), lambda i: (0, i, 0)),
            pl.BlockSpec((_RB, F), lambda i: (i, 0)),
        ],
        out_specs=pl.BlockSpec((_RB, F), lambda i: (i, 0)),
        out_shape=jax.ShapeDtypeStruct((N, F), jnp.float32),
    )(parts, degp, root)


def kernel(x, edge_index, init_weight, root_weight, bias):
    row = edge_index[0]
    col = edge_index[1]
    deg2, _ = _sc_degree(col)
    degp = deg2[:, :N, None]
    b2 = bias[0, 0]
    h2, root = _tc_matmuls(x, degp, init_weight[0], root_weight[0, 0], b2)
    parts = _sc_propagate(h2, row, col)
    return _tc_finish(parts, degp, root)


# speedup vs baseline: 11.8845x; 1.0020x over previous
import dataclasses
import functools

import jax
import jax.numpy as jnp
from jax import lax
from jax.experimental import pallas as pl
from jax.experimental.pallas import tpu as pltpu
from jax.experimental.pallas import tpu_sc as plsc

N = 10000
E = 320000
F = 128

NC = 2
NS = 16
NW = NC * NS

EPT = E // NW
NPAD = 10240
SLICE = NPAD // NS

_vector_mesh = plsc.VectorSubcoreMesh(
    core_axis_name="c", subcore_axis_name="s")

_sc_params = pltpu.CompilerParams()
if "needs_layout_passes" in pltpu.CompilerParams.__dataclass_fields__:
    _sc_params = dataclasses.replace(_sc_params, needs_layout_passes=False)


@functools.partial(
    pl.kernel,
    out_type=[
        jax.ShapeDtypeStruct((NC, NPAD), jnp.float32),
        jax.ShapeDtypeStruct((NC, NS, NPAD), jnp.float32),
    ],
    mesh=_vector_mesh,
    compiler_params=_sc_params,
    scratch_types=[
        pltpu.VMEM((NPAD,), jnp.float32),
        pltpu.VMEM((EPT,), jnp.int32),
        pltpu.VMEM((SLICE,), jnp.float32),
        pltpu.VMEM((NS, SLICE), jnp.float32),
    ],
)
def _sc_degree(col_hbm, out_hbm, stage_hbm, hist_v, cidx_v, acc_v, tmp_v):
    cid = lax.axis_index("c")
    sid = lax.axis_index("s")
    wid = cid * NS + sid

    @pl.loop(0, NPAD, step=16)
    def _zero(i):
        hist_v[pl.ds(i, 16)] = jnp.zeros((16,), jnp.float32)

    pltpu.sync_copy(col_hbm.at[pl.ds(wid * EPT, EPT)], cidx_v)
    ones = jnp.ones((16,), jnp.float32)

    @pl.loop(0, EPT, step=16)
    def _hist(i):
        idx = cidx_v[pl.ds(i, 16)]
        plsc.addupdate_scatter(hist_v, [idx], ones)

    pltpu.sync_copy(hist_v, stage_hbm.at[cid, sid])
    plsc.subcore_barrier()

    base = sid * SLICE
    pltpu.sync_copy(stage_hbm.at[cid, :, pl.ds(base, SLICE)], tmp_v)

    @pl.loop(0, SLICE, step=16)
    def _add(j):
        s = tmp_v[0, pl.ds(j, 16)]
        for k in range(1, NS):
            s = s + tmp_v[k, pl.ds(j, 16)]
        acc_v[pl.ds(j, 16)] = s

    pltpu.sync_copy(acc_v, out_hbm.at[cid, pl.ds(base, SLICE)])


ECH = 128
EPC = E // NS
SCH = 4000
NSC = EPC // SCH
KMAX = 4352
NHALF = 5120
NP2 = 2 * NHALF
TRASH = NHALF
RPT = NHALF // NS
DB = 160


@functools.partial(
    pl.kernel,
    out_type=jax.ShapeDtypeStruct((NP2, F), jnp.float32),
    mesh=_vector_mesh,
    compiler_params=_sc_params,
    scratch_types=[
        pltpu.VMEM((KMAX,), jnp.int32),
        pltpu.VMEM((KMAX // ECH, ECH), jnp.int32),
        pltpu.VMEM((SCH,), jnp.int32),
        pltpu.VMEM((SCH,), jnp.int32),
        pltpu.VMEM((2, ECH, F), jnp.float32),
        pltpu.VMEM((DB, F), jnp.float32),
        pltpu.SemaphoreType.DMA,
        pltpu.SemaphoreType.DMA,
        pltpu.SemaphoreType.DMA,
        pltpu.SemaphoreType.DMA,
        pltpu.VMEM_SHARED((NHALF + 8, F), jnp.float32),
    ],
)
def _sc_propagate(h2_hbm, row_hbm, col_hbm, out_hbm,
                  rkeep, ckeep, escan_r, escan_c, rows_v, dump_v,
                  sg0, sg1, ss0, ss1, acc_sp):
    cid = lax.axis_index("c")
    sid = lax.axis_index("s")
    ebase = sid * EPC
    rbase = sid * RPT
    lo = cid * NHALF

    @pl.loop(0, DB)
    def _zr(r):
        @pl.loop(0, F, step=16)
        def _zq(q):
            dump_v[r, pl.ds(q, 16)] = jnp.zeros((16,), jnp.float32)

    pltpu.sync_copy(dump_v, acc_sp.at[pl.ds(rbase, DB)])
    pltpu.sync_copy(dump_v, acc_sp.at[pl.ds(rbase + DB, DB)])

    @pl.when(sid == 0)
    def _zero_trash():
        pltpu.sync_copy(dump_v.at[pl.ds(0, 8)], acc_sp.at[pl.ds(NHALF, 8)])

    plsc.subcore_barrier()

    for p in range(NSC):
        e0 = ebase + p * SCH
        pltpu.sync_copy(row_hbm.at[pl.ds(e0, SCH)], escan_r)
        pltpu.sync_copy(col_hbm.at[pl.ds(e0, SCH)], escan_c)

        def _scan(i, offv):
            r16 = escan_r[pl.ds(i, 16)]
            c16 = escan_c[pl.ds(i, 16)]
            loc = c16 - lo
            ok = (loc >= 0) & (loc < NHALF)
            pcnt = plsc.all_reduce_population_count(ok)
            cs = plsc.cumsum(jnp.where(ok, 1, 0).astype(jnp.int32))
            pos = offv + cs - 1
            plsc.store_scatter(rkeep, [pos], r16, mask=ok)
            plsc.store_scatter(ckeep, [pos >> 7, pos & 127], loc, mask=ok)
            return offv + pcnt

        offv = pl.loop(0, SCH, step=16, unroll=2,
                       init_carry=jnp.zeros((16,), jnp.int32))(_scan)
        kc = jnp.max(offv)

        @pl.loop(0, 256, step=16)
        def _pad(t):
            idx16 = kc + t + lax.iota(jnp.int32, 16)
            plsc.store_scatter(rkeep, [idx16], jnp.zeros((16,), jnp.int32))
            plsc.store_scatter(ckeep, [idx16 >> 7, idx16 & 127],
                               jnp.full((16,), TRASH, jnp.int32))

        nch = (kc + (ECH - 1)) // ECH

        @pl.when(nch > 0)
        def _prologue():
            pltpu.async_copy(h2_hbm.at[rkeep.at[pl.ds(0, ECH)]],
                             rows_v.at[0], sg0)

        def _item(item, b, sg, sg_o, ss, ss_o):
            pltpu.make_async_copy(
                h2_hbm.at[rkeep.at[pl.ds(item * ECH, ECH)]],
                rows_v.at[b], sg).wait()

            @pl.when((item >= 1) & (item + 1 < nch))
            def _wait_prev():
                pltpu.make_async_copy(rows_v.at[1 - b],
                                      acc_sp.at[ckeep.at[item - 1]],
                                      ss_o).wait()

            @pl.when(item + 1 < nch)
            def _next():
                pltpu.async_copy(
                    h2_hbm.at[rkeep.at[pl.ds((item + 1) * ECH, ECH)]],
                    rows_v.at[1 - b], sg_o)

            pltpu.async_copy(rows_v.at[b], acc_sp.at[ckeep.at[item]], ss,
                             add=True)

        @pl.loop(0, nch, step=2)
        def _phase_b(j):
            _item(j, 0, sg0, sg1, ss0, ss1)

            @pl.when(j + 1 < nch)
            def _odd():
                _item(j + 1, 1, sg1, sg0, ss1, ss0)

        @pl.when(nch >= 2)
        def _drain2():
            pltpu.make_async_copy(rows_v.at[0], acc_sp.at[ckeep.at[0]],
                                  ss0).wait()
            pltpu.make_async_copy(rows_v.at[1], acc_sp.at[ckeep.at[0]],
                                  ss1).wait()

        @pl.when(nch == 1)
        def _drain1():
            pltpu.make_async_copy(rows_v.at[0], acc_sp.at[ckeep.at[0]],
                                  ss0).wait()

    plsc.subcore_barrier()
    for t in range(2):
        pltpu.sync_copy(acc_sp.at[pl.ds(rbase + t * DB, DB)], dump_v)
        pltpu.sync_copy(dump_v, out_hbm.at[pl.ds(lo + rbase + t * DB, DB)])


_RB = 1000
_NB = N // _RB


def _tc_mm_body(x_ref, dg_ref, wi_ref, wr_ref, b_ref, h2_ref, root_ref):
    d = dg_ref[0] + dg_ref[1]
    dis = jnp.where(d > 0, lax.rsqrt(d), 0.0)
    xb = x_ref[...]
    h2_ref[...] = jnp.dot(xb, wi_ref[...],
                          preferred_element_type=jnp.float32) * dis
    root_ref[...] = jnp.dot(xb, wr_ref[...],
                            preferred_element_type=jnp.float32) + b_ref[...]


def _tc_matmuls(x, degp, wi, wr, b2):
    return pl.pallas_call(
        _tc_mm_body,
        grid=(_NB,),
        in_specs=[
            pl.BlockSpec((_RB, F), lambda i: (i, 0)),
            pl.BlockSpec((NC, _RB, 1), lambda i: (0, i, 0)),
            pl.BlockSpec((F, F), lambda i: (0, 0)),
            pl.BlockSpec((F, F), lambda i: (0, 0)),
            pl.BlockSpec((1, F), lambda i: (0, 0)),
        ],
        out_specs=[
            pl.BlockSpec((_RB, F), lambda i: (i, 0)),
            pl.BlockSpec((_RB, F), lambda i: (i, 0)),
        ],
        out_shape=[
            jax.ShapeDtypeStruct((N, F), jnp.float32),
            jax.ShapeDtypeStruct((N, F), jnp.float32),
        ],
    )(x, degp, wi, wr, b2)


def _tc_fin_body(p_ref, dg_ref, root_ref, o_ref):
    d = dg_ref[0] + dg_ref[1]
    dis = jnp.where(d > 0, lax.rsqrt(d), 0.0)
    s = p_ref[...] * dis + root_ref[...]
    o_ref[...] = jnp.maximum(s, 0.0)


def _tc_finish(parts, degp, root):
    return pl.pallas_call(
        _tc_fin_body,
        grid=(_NB,),
        in_specs=[
            pl.BlockSpec((_RB, F), lambda i: (i, 0)),
            pl.BlockSpec((NC, _RB, 1), lambda i: (0, i, 0)),
            pl.BlockSpec((_RB, F), lambda i: (i, 0)),
        ],
        out_specs=pl.BlockSpec((_RB, F), lambda i: (i, 0)),
        out_shape=jax.ShapeDtypeStruct((N, F), jnp.float32),
    )(parts, degp, root)


def kernel(x, edge_index, init_weight, root_weight, bias):
    row = edge_index[0]
    col = edge_index[1]
    deg2, _ = _sc_degree(col)
    degp = deg2[:, :N, None]
    b2 = bias[0, 0]
    h2, root = _tc_matmuls(x, degp, init_weight[0], root_weight[0, 0], b2)
    parts = _sc_propagate(h2, row, col)
    return _tc_finish(parts, degp, root)
